# baseline (speedup 1.0000x reference)
import functools

import jax
import jax.numpy as jnp
from jax import lax
from jax.experimental import pallas as pl
from jax.experimental.pallas import tpu as pltpu
from jax.experimental.pallas import tpu_sc as plsc

N_CORES = 2
N_SUBCORES = 16
N_WORKERS = N_CORES * N_SUBCORES
GROUP = 16
BLK = 128


def _lookup_kernel(B, V, D):
  b_per_w = B // N_WORKERS
  n_groups = b_per_w // GROUP
  mesh = plsc.VectorSubcoreMesh(core_axis_name="c", subcore_axis_name="s")

  @functools.partial(
      pl.kernel,
      out_type=jax.ShapeDtypeStruct((D, B), jnp.float32),
      mesh=mesh,
      scratch_types=[
          pltpu.VMEM((b_per_w,), jnp.int32),
          pltpu.VMEM((GROUP * D, BLK), jnp.float32),
          pltpu.VMEM((D, b_per_w), jnp.float32),
          pltpu.SemaphoreType.DMA,
      ],
      compiler_params=pltpu.CompilerParams(needs_layout_passes=False),
  )
  def run(idx_hbm, table_hbm, out_hbm, idx_v, ring, out_stage, sem):
    wid = lax.axis_index("s") * N_CORES + lax.axis_index("c")
    base = wid * b_per_w
    pltpu.sync_copy(idx_hbm.at[pl.ds(base, b_per_w)], idx_v)
    iota = lax.iota(jnp.int32, GROUP)
    svec = iota * D

    def body(g, carry):
      idxvec = idx_v[pl.ds(g * GROUP, GROUP)]
      offvec = lax.bitwise_and(idxvec, -BLK)
      lvec = lax.bitwise_and(idxvec, BLK - 1)
      for k in range(GROUP):
        off = pl.multiple_of(offvec[k], BLK)
        pltpu.async_copy(
            table_hbm.at[:, pl.ds(off, BLK)],
            ring.at[pl.ds(k * D, D), :],
            sem,
        )
      for k in range(GROUP):
        pltpu.make_async_copy(
            table_hbm.at[:, pl.ds(0, BLK)],
            ring.at[pl.ds(k * D, D), :],
            sem,
        ).wait()
      for d in range(D):
        vals = plsc.load_gather(ring, [svec + d, lvec])
        out_stage[d, pl.ds(g * GROUP, GROUP)] = vals
      return carry

    lax.fori_loop(0, n_groups, body, 0)
    pltpu.sync_copy(out_stage, out_hbm.at[:, pl.ds(base, b_per_w)])

  return run


def kernel(idx, emb_weight):
  B = idx.shape[0]
  V, D = emb_weight.shape
  run = _lookup_kernel(B, V, D)
  out_t = run(idx.astype(jnp.int32), emb_weight.T)
  return out_t.T
